# baseline (speedup 1.0000x reference)
import functools

import jax
import jax.numpy as jnp
from jax import lax
from jax.experimental import pallas as pl
from jax.experimental.pallas import tpu as pltpu
from jax.experimental.pallas import tpu_sc as plsc

N = 10000
E = 320000
D = 128

NC = 2
NS = 16
NW = NC * NS
K = 128
NBLK = E // K
BPT = NBLK // NW
NPAD = NS * 640
RPT = NPAD // NS


def _sc_body(x_hbm, src_hbm, dst_hbm, acc_out, cnt_out,
             sidx0, sidx1, sidx2, sidx3, didx0, didx1, didx2, didx3,
             rows_v, ones_v, zcnt_v, acc_sh, cnt_sh,
             isem0, isem1, isem2, isem3, gsem0, gsem1, ssem0, ssem1):
    c = lax.axis_index("c")
    s = lax.axis_index("s")
    w = s * NC + c

    zeros16 = jnp.zeros((16,), jnp.float32)
    ones16 = jnp.ones((16,), jnp.float32)

    def _fill(r, carry):
        for j in range(D // 16):
            rows_v[0, r, pl.ds(j * 16, 16)] = zeros16
        return carry

    lax.fori_loop(0, K, _fill, 0)

    def _fill1(r, carry):
        ones_v[pl.ds(r * 16, 16)] = ones16
        return carry

    lax.fori_loop(0, K // 16, _fill1, 0)

    def _fillz(r, carry):
        zcnt_v[pl.ds(r * 16, 16)] = zeros16
        return carry

    lax.fori_loop(0, RPT // 16, _fillz, 0)

    rbase = pl.multiple_of(s * RPT, 8)
    for t in range(RPT // K):
        pltpu.sync_copy(rows_v.at[0], acc_sh.at[pl.ds(rbase + t * K, K), :])
    pltpu.sync_copy(zcnt_v, cnt_sh.at[pl.ds(rbase, RPT)])
    plsc.subcore_barrier()

    sidxs = (sidx0, sidx1, sidx2, sidx3)
    didxs = (didx0, didx1, didx2, didx3)
    isems = (isem0, isem1, isem2, isem3)
    gsems = (gsem0, gsem1)
    ssems = (ssem0, ssem1)

    def _load_idx(i, q):
        blk = i * NW + w
        pltpu.async_copy(src_hbm.at[blk], sidxs[q], isems[q])
        pltpu.async_copy(dst_hbm.at[blk], didxs[q], isems[q])

    def _wait_idx(q):
        pltpu.make_async_copy(src_hbm.at[0], sidxs[q], isems[q]).wait()
        pltpu.make_async_copy(dst_hbm.at[0], didxs[q], isems[q]).wait()

    def _wait_scatter(b):
        pltpu.make_async_copy(
            rows_v.at[b], acc_sh.at[didxs[b]], ssems[b]).wait()

    nbt = BPT + jnp.where(w < NBLK - BPT * NW, 1, 0)

    _load_idx(0, 0)
    _load_idx(1, 1)
    _wait_idx(0)
    pltpu.async_copy(x_hbm.at[sidx0], rows_v.at[0], gsem0)

    def _chunk(j, carry):
        for q in range(4):
            i = 4 * j + q
            b = q % 2
            nb = 1 - b

            @pl.when(i < nbt)
            def _():
                pltpu.make_async_copy(
                    x_hbm.at[sidxs[q]], rows_v.at[b], gsems[b]).wait()

            @pl.when(i + 1 < nbt)
            def _():
                _wait_idx((q + 1) % 4)
                pltpu.async_copy(x_hbm.at[sidxs[(q + 1) % 4]],
                                 rows_v.at[nb], gsems[nb])

            @pl.when(i < nbt)
            def _():
                pltpu.sync_copy(rows_v.at[b], acc_sh.at[didxs[q]], add=True)
                pltpu.sync_copy(ones_v, cnt_sh.at[didxs[q]], add=True)

            @pl.when(i + 2 < nbt)
            def _():
                _load_idx(i + 2, (q + 2) % 4)
        return carry

    lax.fori_loop(0, (BPT + 4) // 4, _chunk, 0)

    plsc.subcore_barrier()

    pltpu.sync_copy(acc_sh.at[pl.ds(rbase, RPT)],
                    acc_out.at[c, pl.ds(rbase, RPT), :])
    pltpu.sync_copy(cnt_sh.at[pl.ds(rbase, RPT)],
                    cnt_out.at[c, pl.ds(rbase, RPT)])


_sc_scatter = functools.partial(
    pl.kernel,
    mesh=plsc.VectorSubcoreMesh(core_axis_name="c", subcore_axis_name="s"),
    out_type=[
        jax.ShapeDtypeStruct((NC, NPAD, D), jnp.float32),
        jax.ShapeDtypeStruct((NC, NPAD), jnp.float32),
    ],
    scratch_types=[
        pltpu.VMEM((K,), jnp.int32),
        pltpu.VMEM((K,), jnp.int32),
        pltpu.VMEM((K,), jnp.int32),
        pltpu.VMEM((K,), jnp.int32),
        pltpu.VMEM((K,), jnp.int32),
        pltpu.VMEM((K,), jnp.int32),
        pltpu.VMEM((K,), jnp.int32),
        pltpu.VMEM((K,), jnp.int32),
        pltpu.VMEM((2, K, D), jnp.float32),
        pltpu.VMEM((K,), jnp.float32),
        pltpu.VMEM((RPT,), jnp.float32),
        pltpu.VMEM_SHARED((NPAD, D), jnp.float32),
        pltpu.VMEM_SHARED((NPAD,), jnp.float32),
        pltpu.SemaphoreType.DMA,
        pltpu.SemaphoreType.DMA,
        pltpu.SemaphoreType.DMA,
        pltpu.SemaphoreType.DMA,
        pltpu.SemaphoreType.DMA,
        pltpu.SemaphoreType.DMA,
        pltpu.SemaphoreType.DMA,
        pltpu.SemaphoreType.DMA,
    ],
)(_sc_body)


BLK = 512


def _tc_root_body(x_ref, wr_ref, b_ref, o_ref):
    x = x_ref[...]
    o_ref[...] = (x + jnp.dot(x, wr_ref[...],
                              preferred_element_type=jnp.float32)
                  + b_ref[...])


def _tc_root(x, wr_t, b_row):
    return pl.pallas_call(
        _tc_root_body,
        grid=(pl.cdiv(N, BLK),),
        in_specs=[
            pl.BlockSpec((BLK, D), lambda i: (i, 0)),
            pl.BlockSpec((D, D), lambda i: (0, 0)),
            pl.BlockSpec((1, D), lambda i: (0, 0)),
        ],
        out_specs=pl.BlockSpec((BLK, D), lambda i: (i, 0)),
        out_shape=jax.ShapeDtypeStruct((N, D), jnp.float32),
    )(x, wr_t, b_row)


def _tc_body(r_ref, p_ref, c_ref, wl_ref, o_ref):
    p = p_ref[0] + p_ref[1]
    cnt = c_ref[0] + c_ref[1]
    mean = p / jnp.maximum(cnt, 1.0)[:, None]
    o_ref[...] = r_ref[...] + jnp.dot(mean, wl_ref[...],
                                      preferred_element_type=jnp.float32)


def _tc_finish(r, acc, cnt, wl_t):
    return pl.pallas_call(
        _tc_body,
        grid=(pl.cdiv(N, BLK),),
        in_specs=[
            pl.BlockSpec((BLK, D), lambda i: (i, 0)),
            pl.BlockSpec((NC, BLK, D), lambda i: (0, i, 0)),
            pl.BlockSpec((NC, BLK), lambda i: (0, i)),
            pl.BlockSpec((D, D), lambda i: (0, 0)),
        ],
        out_specs=pl.BlockSpec((BLK, D), lambda i: (i, 0)),
        out_shape=jax.ShapeDtypeStruct((N, D), jnp.float32),
    )(r, acc, cnt, wl_t)


def kernel(x, edge_index, W_l, b_l, W_r):
    x = x.astype(jnp.float32)
    ei = edge_index.astype(jnp.int32)
    src = ei[0].reshape(NBLK, K)
    dst = ei[1].reshape(NBLK, K)
    acc, cnt = _sc_scatter(x, src, dst)
    r = _tc_root(x, W_r.T, b_l.reshape(1, D))
    return _tc_finish(r, acc, cnt, W_l.T)


# speedup vs baseline: 1.0305x; 1.0305x over previous
import functools

import jax
import jax.numpy as jnp
from jax import lax
from jax.experimental import pallas as pl
from jax.experimental.pallas import tpu as pltpu
from jax.experimental.pallas import tpu_sc as plsc

N = 10000
E = 320000
D = 128

NC = 2
NS = 16
NW = NC * NS
K = 128
NBLK = 2560
EPAD = NBLK * K
BPT = NBLK // NW
NPAD = NS * 640
RPT = NPAD // NS


def _sc_body(x_hbm, src_hbm, dst_hbm, acc_out, cnt_out,
             sidx0, sidx1, didx0, didx1, rows_v, ones_v, zcnt_v,
             acc_sh, cnt_sh, isem0, isem1, gsem0, gsem1):
    c = lax.axis_index("c")
    s = lax.axis_index("s")
    w = s * NC + c

    zeros16 = jnp.zeros((16,), jnp.float32)
    ones16 = jnp.ones((16,), jnp.float32)

    def _fill(r, carry):
        for j in range(D // 16):
            rows_v[0, r, pl.ds(j * 16, 16)] = zeros16
        return carry

    lax.fori_loop(0, K, _fill, 0)

    def _fill1(r, carry):
        ones_v[pl.ds(r * 16, 16)] = ones16
        return carry

    lax.fori_loop(0, K // 16, _fill1, 0)

    def _fillz(r, carry):
        zcnt_v[pl.ds(r * 16, 16)] = zeros16
        return carry

    lax.fori_loop(0, RPT // 16, _fillz, 0)

    rbase = pl.multiple_of(s * RPT, 8)
    for t in range(RPT // K):
        pltpu.sync_copy(rows_v.at[0], acc_sh.at[pl.ds(rbase + t * K, K), :])
    pltpu.sync_copy(zcnt_v, cnt_sh.at[pl.ds(rbase, RPT)])
    plsc.subcore_barrier()

    sidxs = (sidx0, sidx1)
    didxs = (didx0, didx1)
    isems = (isem0, isem1)
    gsems = (gsem0, gsem1)

    def _load_idx(i, b):
        blk = i * NW + w
        pltpu.async_copy(src_hbm.at[blk], sidxs[b], isems[b])
        pltpu.async_copy(dst_hbm.at[blk], didxs[b], isems[b])

    def _wait_idx(b):
        pltpu.make_async_copy(src_hbm.at[0], sidxs[b], isems[b]).wait()
        pltpu.make_async_copy(dst_hbm.at[0], didxs[b], isems[b]).wait()

    _load_idx(0, 0)
    _load_idx(1, 1)
    _wait_idx(0)
    pltpu.async_copy(x_hbm.at[sidx0], rows_v.at[0], gsem0)

    def _chunk(j, carry):
        for b in range(2):
            i = 2 * j + b
            nb = 1 - b

            @pl.when(i + 1 < BPT)
            def _():
                _wait_idx(nb)
                pltpu.async_copy(x_hbm.at[sidxs[nb]], rows_v.at[nb],
                                 gsems[nb])

            pltpu.make_async_copy(
                x_hbm.at[sidxs[b]], rows_v.at[b], gsems[b]).wait()
            pltpu.sync_copy(rows_v.at[b], acc_sh.at[didxs[b]], add=True)
            pltpu.sync_copy(ones_v, cnt_sh.at[didxs[b]], add=True)

            @pl.when(i + 2 < BPT)
            def _():
                _load_idx(i + 2, b)
        return carry

    lax.fori_loop(0, BPT // 2, _chunk, 0)

    plsc.subcore_barrier()

    pltpu.sync_copy(acc_sh.at[pl.ds(rbase, RPT)],
                    acc_out.at[c, pl.ds(rbase, RPT), :])
    pltpu.sync_copy(cnt_sh.at[pl.ds(rbase, RPT)],
                    cnt_out.at[c, pl.ds(rbase, RPT)])


_sc_scatter = functools.partial(
    pl.kernel,
    mesh=plsc.VectorSubcoreMesh(core_axis_name="c", subcore_axis_name="s"),
    out_type=[
        jax.ShapeDtypeStruct((NC, NPAD, D), jnp.float32),
        jax.ShapeDtypeStruct((NC, NPAD), jnp.float32),
    ],
    scratch_types=[
        pltpu.VMEM((K,), jnp.int32),
        pltpu.VMEM((K,), jnp.int32),
        pltpu.VMEM((K,), jnp.int32),
        pltpu.VMEM((K,), jnp.int32),
        pltpu.VMEM((2, K, D), jnp.float32),
        pltpu.VMEM((K,), jnp.float32),
        pltpu.VMEM((RPT,), jnp.float32),
        pltpu.VMEM_SHARED((NPAD, D), jnp.float32),
        pltpu.VMEM_SHARED((NPAD,), jnp.float32),
        pltpu.SemaphoreType.DMA,
        pltpu.SemaphoreType.DMA,
        pltpu.SemaphoreType.DMA,
        pltpu.SemaphoreType.DMA,
    ],
)(_sc_body)


BLK = 512


def _tc_body(x_ref, p_ref, c_ref, wl_ref, wr_ref, b_ref, o_ref):
    x = x_ref[...]
    p = p_ref[0] + p_ref[1]
    cnt = c_ref[0] + c_ref[1]
    mean = p / jnp.maximum(cnt, 1.0)[:, None]
    y = (jnp.dot(mean, wl_ref[...], preferred_element_type=jnp.float32)
         + jnp.dot(x, wr_ref[...], preferred_element_type=jnp.float32)
         + b_ref[...])
    o_ref[...] = x + y


def _tc_finish(x, acc, cnt, wl_t, wr_t, b_row):
    return pl.pallas_call(
        _tc_body,
        grid=(pl.cdiv(N, BLK),),
        in_specs=[
            pl.BlockSpec((BLK, D), lambda i: (i, 0)),
            pl.BlockSpec((NC, BLK, D), lambda i: (0, i, 0)),
            pl.BlockSpec((NC, BLK), lambda i: (0, i)),
            pl.BlockSpec((D, D), lambda i: (0, 0)),
            pl.BlockSpec((D, D), lambda i: (0, 0)),
            pl.BlockSpec((1, D), lambda i: (0, 0)),
        ],
        out_specs=pl.BlockSpec((BLK, D), lambda i: (i, 0)),
        out_shape=jax.ShapeDtypeStruct((N, D), jnp.float32),
    )(x, acc, cnt, wl_t, wr_t, b_row)


def kernel(x, edge_index, W_l, b_l, W_r):
    x = x.astype(jnp.float32)
    ei = edge_index.astype(jnp.int32)
    npad_e = EPAD - E
    pad_iota = jnp.arange(npad_e, dtype=jnp.int32)
    src = jnp.concatenate(
        [ei[0], pad_iota % N]).reshape(NBLK, K)
    dst = jnp.concatenate(
        [ei[1], N + pad_iota % (NPAD - N)]).reshape(NBLK, K)
    acc, cnt = _sc_scatter(x, src, dst)
    return _tc_finish(x, acc, cnt, W_l.T, W_r.T, b_l.reshape(1, D))
